# baseline (speedup 1.0000x reference)
import functools

import jax
import jax.numpy as jnp
from jax import lax
from jax.experimental import pallas as pl
from jax.experimental.pallas import tpu as pltpu
from jax.experimental.pallas import tpu_sc as plsc

_B = 4096
_GT, _GN, _GD = 13, 100000, 64
_CT, _CN, _CD = 13, 1000000, 32

_NC, _NS = 2, 16
_NW = _NC * _NS
_ROWS = _B * _GT
_RPW = _ROWS // _NW
_CHUNK = 128
_NCHUNK = _RPW // _CHUNK


def _sc_gather(gt_flat, gidx2d, ct_flat, cidx2d):
    mesh = plsc.VectorSubcoreMesh(
        core_axis_name="c", subcore_axis_name="s",
        num_cores=_NC, num_subcores=_NS)

    @functools.partial(
        pl.kernel,
        out_type=(jax.ShapeDtypeStruct((_ROWS, _GD), jnp.float32),
                  jax.ShapeDtypeStruct((_ROWS, _CD), jnp.float32)),
        mesh=mesh,
        scratch_types=(
            pltpu.VMEM((_NCHUNK, _CHUNK), jnp.int32),
            pltpu.VMEM((_CHUNK, _GD), jnp.float32),
            pltpu.VMEM((_NCHUNK, _CHUNK), jnp.int32),
            pltpu.VMEM((_CHUNK, _CD), jnp.float32),
            pltpu.SemaphoreType.DMA,
        ),
        compiler_params=pltpu.CompilerParams(use_tc_tiling_on_sc=False),
    )
    def k(gt_hbm, gidx_hbm, ct_hbm, cidx_hbm, gout_hbm, cout_hbm,
          gi_v, gr_v, ci_v, cr_v, sem):
        wid = lax.axis_index("s") * _NC + lax.axis_index("c")
        rbase = wid * _RPW
        pltpu.sync_copy(gidx_hbm.at[wid], gi_v)
        pltpu.sync_copy(cidx_hbm.at[wid], ci_v)
        for j in range(_NCHUNK):
            pltpu.async_copy(gt_hbm.at[gi_v.at[j]], gr_v, sem).wait()
            pltpu.sync_copy(gr_v, gout_hbm.at[pl.ds(rbase + j * _CHUNK, _CHUNK)])
        for j in range(_NCHUNK):
            pltpu.async_copy(ct_hbm.at[ci_v.at[j]], cr_v, sem).wait()
            pltpu.sync_copy(cr_v, cout_hbm.at[pl.ds(rbase + j * _CHUNK, _CHUNK)])

    return k(gt_flat, gidx2d, ct_flat, cidx2d)


def _mlp_body(df, ge, ce, dw1t, db1, dw2t, db2,
              w1gt, w1ct, w1dt, ob1, ow2t, ob2, ow3t, ob3, ow4t, ob4,
              ow5t, ob5, out):
    dot = functools.partial(jnp.dot, preferred_element_type=jnp.float32)
    h = jnp.maximum(dot(df[...], dw1t[...]) + db1[...], 0.0)
    de = dot(h, dw2t[...]) + db2[...]
    o = dot(ge[...], w1gt[...]) + dot(ce[...], w1ct[...]) + dot(de, w1dt[...])
    o = jnp.maximum(o + ob1[...], 0.0)
    o = jnp.maximum(dot(o, ow2t[...]) + ob2[...], 0.0)
    o = jnp.maximum(dot(o, ow3t[...]) + ob3[...], 0.0)
    o = jnp.maximum(dot(o, ow4t[...]) + ob4[...], 0.0)
    out[...] = dot(o, ow5t[...]) + ob5[...]


def _tc_mlp(df, ge, ce, dw1t, db1, dw2t, db2,
            w1gt, w1ct, w1dt, ob1, ow2t, ob2, ow3t, ob3, ow4t, ob4,
            ow5t, ob5, block_b=512):
    grid = (_B // block_b,)

    def row_spec(cols):
        return pl.BlockSpec((block_b, cols), lambda i: (i, 0))

    def full_spec(a):
        return pl.BlockSpec(a.shape, lambda i: (0,) * a.ndim)

    weights = (dw1t, db1, dw2t, db2, w1gt, w1ct, w1dt, ob1,
               ow2t, ob2, ow3t, ob3, ow4t, ob4, ow5t, ob5)
    return pl.pallas_call(
        _mlp_body,
        grid=grid,
        in_specs=[row_spec(df.shape[1]), row_spec(ge.shape[1]),
                  row_spec(ce.shape[1])] + [full_spec(w) for w in weights],
        out_specs=pl.BlockSpec((block_b, 1), lambda i: (i, 0)),
        out_shape=jax.ShapeDtypeStruct((_B, 1), jnp.float32),
    )(df, ge, ce, *weights)


def kernel(dense_features, gpu_sharded_sparse_features, cpu_sharded_sparse_features,
           gpu_tables, cpu_tables, dw1, db1, dw2, db2,
           ow1, ob1, ow2, ob2, ow3, ob3, ow4, ob4, ow5, ob5):
    gidx = (gpu_sharded_sparse_features.astype(jnp.int32)
            + jnp.arange(_GT, dtype=jnp.int32)[None, :] * _GN)
    cidx = (cpu_sharded_sparse_features.astype(jnp.int32)
            + jnp.arange(_CT, dtype=jnp.int32)[None, :] * _CN)
    gidx2d = gidx.reshape(_NW, _NCHUNK, _CHUNK)
    cidx2d = cidx.reshape(_NW, _NCHUNK, _CHUNK)

    grows, crows = _sc_gather(
        gpu_tables.reshape(_GT * _GN, _GD), gidx2d,
        cpu_tables.reshape(_CT * _CN, _CD), cidx2d)
    ge = grows.reshape(_B, _GT * _GD)
    ce = crows.reshape(_B, _CT * _CD)

    g_cols = _GT * _GD
    c_cols = _CT * _CD
    ow1t = ow1.T
    w1gt = ow1t[:g_cols]
    w1ct = ow1t[g_cols:g_cols + c_cols]
    w1dt = ow1t[g_cols + c_cols:]

    return _tc_mlp(
        dense_features, ge, ce,
        dw1.T, db1[None, :], dw2.T, db2[None, :],
        w1gt, w1ct, w1dt, ob1[None, :],
        ow2.T, ob2[None, :], ow3.T, ob3[None, :], ow4.T, ob4[None, :],
        ow5.T, ob5[None, :])
